# baseline (device time: 22264 ns/iter reference)
import jax
import jax.numpy as jnp
from jax import lax
from jax.experimental import pallas as pl
from jax.experimental.pallas import tpu as pltpu

Z = 4
H = 4


def kernel(x):
    m, n = x.shape
    blk = n // Z
    qm = m // 4
    hm = qm // H

    def body(x_ref, out_ref, xb_ref, zs_sems, zr_sems, ps_sems, pr_sems):
        my_x = lax.axis_index("x")
        my_y = lax.axis_index("y")
        my_z = lax.axis_index("z")
        q = 2 * my_y + my_x

        for d in range(1, Z):
            tz = (my_z + d) % Z
            for h in range(H):
                xb_ref[(d - 1) * H + h] = x_ref[
                    pl.ds(q * qm + h * hm, hm), pl.ds(tz * blk, blk)
                ].astype(jnp.bfloat16)

        peers = (
            (1 - my_x, 1 - my_y, my_z),
            (1 - my_x, my_y, my_z),
            (my_x, 1 - my_y, my_z),
        )

        barrier_sem = pltpu.get_barrier_semaphore()
        for d in range(1, Z):
            pl.semaphore_signal(
                barrier_sem, inc=1,
                device_id=(my_x, my_y, (my_z + d) % Z),
                device_id_type=pl.DeviceIdType.MESH,
            )
        for nbr in peers:
            pl.semaphore_signal(
                barrier_sem, inc=1,
                device_id=nbr, device_id_type=pl.DeviceIdType.MESH,
            )
        pl.semaphore_wait(barrier_sem, 6)

        z_rdmas = []
        for d in range(1, Z):
            tz = (my_z + d) % Z
            for h in range(H):
                slot = (d - 1) * H + h
                r = pltpu.make_async_remote_copy(
                    src_ref=xb_ref.at[slot],
                    dst_ref=out_ref.at[
                        pl.ds(my_z * m + q * qm + h * hm, hm), :
                    ],
                    send_sem=zs_sems.at[slot],
                    recv_sem=zr_sems.at[slot],
                    device_id=(my_x, my_y, tz),
                    device_id_type=pl.DeviceIdType.MESH,
                )
                r.start()
                z_rdmas.append(r)

        out_ref[pl.ds(my_z * m, m), :] = (
            x_ref[:, pl.ds(my_z * blk, blk)].astype(jnp.bfloat16)
        )

        p_rdmas = []
        for d in range(1, Z):
            sz = (my_z - d) % Z
            for h in range(H):
                zslot = (d - 1) * H + h
                z_rdmas[zslot].wait_recv()
                rows = pl.ds(sz * m + q * qm + h * hm, hm)
                for i, nbr in enumerate(peers):
                    slot = zslot * 3 + i
                    r = pltpu.make_async_remote_copy(
                        src_ref=out_ref.at[rows, :],
                        dst_ref=out_ref.at[rows, :],
                        send_sem=ps_sems.at[slot],
                        recv_sem=pr_sems.at[slot],
                        device_id=nbr,
                        device_id_type=pl.DeviceIdType.MESH,
                    )
                    r.start()
                    p_rdmas.append(r)

        for r in p_rdmas:
            r.wait_recv()
        for r in z_rdmas + p_rdmas:
            r.wait_send()

    out_shape = jax.ShapeDtypeStruct((Z * m, blk), jnp.bfloat16)
    return pl.pallas_call(
        body,
        out_shape=out_shape,
        in_specs=[pl.BlockSpec(memory_space=pltpu.VMEM)],
        out_specs=pl.BlockSpec(memory_space=pltpu.VMEM),
        scratch_shapes=[
            pltpu.VMEM((H * (Z - 1), hm, blk), jnp.bfloat16),
            pltpu.SemaphoreType.DMA((H * (Z - 1),)),
            pltpu.SemaphoreType.DMA((H * (Z - 1),)),
            pltpu.SemaphoreType.DMA((3 * H * (Z - 1),)),
            pltpu.SemaphoreType.DMA((3 * H * (Z - 1),)),
        ],
        compiler_params=pltpu.CompilerParams(collective_id=0),
    )(x)


# device time: 22121 ns/iter; 1.0065x vs baseline; 1.0065x over previous
import jax
import jax.numpy as jnp
from jax import lax
from jax.experimental import pallas as pl
from jax.experimental.pallas import tpu as pltpu

Z = 4
H = 2


def kernel(x):
    m, n = x.shape
    blk = n // Z
    qm = m // 4
    hm = qm // H

    def body(x_ref, out_ref, xb_ref, zs_sems, zr_sems, ps_sems, pr_sems):
        my_x = lax.axis_index("x")
        my_y = lax.axis_index("y")
        my_z = lax.axis_index("z")
        q = 2 * my_y + my_x

        for d in range(1, Z):
            tz = (my_z + d) % Z
            for h in range(H):
                xb_ref[(d - 1) * H + h] = x_ref[
                    pl.ds(q * qm + h * hm, hm), pl.ds(tz * blk, blk)
                ].astype(jnp.bfloat16)

        peers = (
            (1 - my_x, 1 - my_y, my_z),
            (1 - my_x, my_y, my_z),
            (my_x, 1 - my_y, my_z),
        )

        barrier_sem = pltpu.get_barrier_semaphore()
        for d in range(1, Z):
            pl.semaphore_signal(
                barrier_sem, inc=1,
                device_id=(my_x, my_y, (my_z + d) % Z),
                device_id_type=pl.DeviceIdType.MESH,
            )
        for nbr in peers:
            pl.semaphore_signal(
                barrier_sem, inc=1,
                device_id=nbr, device_id_type=pl.DeviceIdType.MESH,
            )
        pl.semaphore_wait(barrier_sem, 6)

        z_rdmas = []
        for d in range(1, Z):
            tz = (my_z + d) % Z
            for h in range(H):
                slot = (d - 1) * H + h
                r = pltpu.make_async_remote_copy(
                    src_ref=xb_ref.at[slot],
                    dst_ref=out_ref.at[
                        pl.ds(my_z * m + q * qm + h * hm, hm), :
                    ],
                    send_sem=zs_sems.at[slot],
                    recv_sem=zr_sems.at[slot],
                    device_id=(my_x, my_y, tz),
                    device_id_type=pl.DeviceIdType.MESH,
                )
                r.start()
                z_rdmas.append(r)

        out_ref[pl.ds(my_z * m, m), :] = (
            x_ref[:, pl.ds(my_z * blk, blk)].astype(jnp.bfloat16)
        )

        p_rdmas = []
        for d in range(1, Z):
            sz = (my_z - d) % Z
            for h in range(H):
                zslot = (d - 1) * H + h
                z_rdmas[zslot].wait_recv()
                rows = pl.ds(sz * m + q * qm + h * hm, hm)
                for i, nbr in enumerate(peers):
                    slot = zslot * 3 + i
                    r = pltpu.make_async_remote_copy(
                        src_ref=out_ref.at[rows, :],
                        dst_ref=out_ref.at[rows, :],
                        send_sem=ps_sems.at[slot],
                        recv_sem=pr_sems.at[slot],
                        device_id=nbr,
                        device_id_type=pl.DeviceIdType.MESH,
                    )
                    r.start()
                    p_rdmas.append(r)

        for r in p_rdmas:
            r.wait_recv()
        for r in z_rdmas + p_rdmas:
            r.wait_send()

    out_shape = jax.ShapeDtypeStruct((Z * m, blk), jnp.bfloat16)
    return pl.pallas_call(
        body,
        out_shape=out_shape,
        in_specs=[pl.BlockSpec(memory_space=pltpu.VMEM)],
        out_specs=pl.BlockSpec(memory_space=pltpu.VMEM),
        scratch_shapes=[
            pltpu.VMEM((H * (Z - 1), hm, blk), jnp.bfloat16),
            pltpu.SemaphoreType.DMA((H * (Z - 1),)),
            pltpu.SemaphoreType.DMA((H * (Z - 1),)),
            pltpu.SemaphoreType.DMA((3 * H * (Z - 1),)),
            pltpu.SemaphoreType.DMA((3 * H * (Z - 1),)),
        ],
        compiler_params=pltpu.CompilerParams(collective_id=0),
    )(x)
